# baseline (speedup 1.0000x reference)
import functools

import jax
import jax.numpy as jnp
from jax import lax
from jax.experimental import pallas as pl
from jax.experimental.pallas import tpu as pltpu
from jax.experimental.pallas import tpu_sc as plsc

BATCH = 16384
HIST = 50
USER_DIM = 128
CAT_DIM = 64
NUM_CATEGORIES = 1000
DENSE_UNITS = 96

NC = 2
NS = 16
NW = NC * NS
BPW = BATCH // NW
LANES = 16
UCH = BPW // 4
NG = BPW // LANES


def _sc_fused(user_ids, user_table, cat_idx_flat, cat_table_flat):
    mesh = plsc.VectorSubcoreMesh(core_axis_name="c", subcore_axis_name="s")
    tab_words = (NUM_CATEGORIES + 1) * CAT_DIM

    @functools.partial(
        pl.kernel,
        out_type=(
            jax.ShapeDtypeStruct((BATCH, USER_DIM), jnp.float32),
            jax.ShapeDtypeStruct((BATCH, CAT_DIM), jnp.float32),
        ),
        mesh=mesh,
        scratch_types=[
            pltpu.VMEM((BPW,), jnp.int32),
            pltpu.VMEM((HIST, BPW), jnp.int32),
            pltpu.VMEM((tab_words,), jnp.float32),
            pltpu.VMEM((2, UCH, USER_DIM), jnp.float32),
            pltpu.VMEM((2, LANES, CAT_DIM), jnp.float32),
            pltpu.SemaphoreType.DMA,
            pltpu.SemaphoreType.DMA,
            pltpu.SemaphoreType.DMA,
        ],
        compiler_params=pltpu.CompilerParams(needs_layout_passes=False),
    )
    def k(uids_hbm, ut_hbm, cidx_hbm, ct_hbm, uout_hbm, cout_hbm,
          uid_v, cidx_v, tab_v, ubuf, obuf, gsem, wsem, csem):
        wid = lax.axis_index("s") * NC + lax.axis_index("c")
        base = wid * BPW
        pltpu.sync_copy(uids_hbm.at[pl.ds(base, BPW)], uid_v)

        def ug_desc(ch):
            return pltpu.make_async_copy(
                ut_hbm.at[uid_v.at[pl.ds(ch * UCH, UCH)]], ubuf.at[ch % 2], gsem)

        def uw_desc(ch):
            return pltpu.make_async_copy(
                ubuf.at[ch % 2], uout_hbm.at[pl.ds(base + ch * UCH, UCH)], wsem)

        ug_desc(0).start()
        pltpu.sync_copy(cidx_hbm.at[:, pl.ds(base, BPW)], cidx_v)
        pltpu.sync_copy(ct_hbm, tab_v)

        lane = lax.iota(jnp.int32, LANES)
        diag = tuple((lane + f0) & (LANES - 1) for f0 in range(LANES))
        inv = jnp.float32(1.0 / HIST)

        def service(ch):
            ug_desc(ch - 1).wait()
            uw_desc(ch - 1).start()
            if ch >= 2:
                uw_desc(ch - 2).wait()
            ug_desc(ch).start()

        def g_body(g, _):
            for ch in (1, 2, 3):
                pl.when(g == 8 * ch)(lambda ch=ch: service(ch))
            gbase = g * LANES
            pl.when(g >= 2)(
                lambda: pltpu.make_async_copy(
                    obuf.at[g % 2],
                    cout_hbm.at[pl.ds(base + gbase - 2 * LANES, LANES)],
                    csem,
                ).wait()
            )
            def dc_body(dc, _):
                foff = dc * LANES

                def h_body(h, acc):
                    c = cidx_v[h, pl.ds(gbase, LANES)]
                    b0 = c * CAT_DIM + foff
                    return tuple(
                        acc[f0] + plsc.load_gather(tab_v, [b0 + diag[f0]])
                        for f0 in range(LANES)
                    )

                acc0 = tuple(jnp.zeros((LANES,), jnp.float32) for _ in range(LANES))
                acc = lax.fori_loop(0, HIST, h_body, acc0, unroll=2)
                for f0 in range(LANES):
                    plsc.store_scatter(
                        obuf.at[g % 2], [lane, foff + diag[f0]], acc[f0] * inv)
                return 0

            lax.fori_loop(0, CAT_DIM // LANES, dc_body, 0)
            pltpu.async_copy(
                obuf.at[g % 2], cout_hbm.at[pl.ds(base + gbase, LANES)], csem)
            return 0

        lax.fori_loop(0, NG, g_body, 0)

        for g in (NG - 2, NG - 1):
            pltpu.make_async_copy(
                obuf.at[g % 2],
                cout_hbm.at[pl.ds(base + g * LANES, LANES)], csem).wait()
        ug_desc(3).wait()
        uw_desc(3).start()
        uw_desc(2).wait()
        uw_desc(3).wait()

    return k(user_ids, user_table, cat_idx_flat, cat_table_flat)


def _mlp(user_emb, cat_mean, W1u, W1c, b1, W2, b2):
    BM = 2048
    grid = (BATCH // BM,)

    def body(u_ref, c_ref, w1u_ref, w1c_ref, b1_ref, w2_ref, b2_ref, o_ref):
        xu = lax.dot_general(
            w1u_ref[...], u_ref[...], (((0,), (1,)), ((), ())),
            preferred_element_type=jnp.float32,
        )
        xc = lax.dot_general(
            w1c_ref[...], c_ref[...], (((0,), (1,)), ((), ())),
            preferred_element_type=jnp.float32,
        )
        x = jax.nn.relu(xu + xc + b1_ref[...])
        z = lax.dot_general(
            w2_ref[...], x, (((0,), (0,)), ((), ())),
            preferred_element_type=jnp.float32,
        )
        o_ref[...] = jax.nn.sigmoid(z + b2_ref[...])

    return pl.pallas_call(
        body,
        grid=grid,
        in_specs=[
            pl.BlockSpec((BM, USER_DIM), lambda i: (i, 0)),
            pl.BlockSpec((BM, CAT_DIM), lambda i: (i, 0)),
            pl.BlockSpec((USER_DIM, DENSE_UNITS), lambda i: (0, 0)),
            pl.BlockSpec((CAT_DIM, DENSE_UNITS), lambda i: (0, 0)),
            pl.BlockSpec((DENSE_UNITS, 1), lambda i: (0, 0)),
            pl.BlockSpec((DENSE_UNITS, NUM_CATEGORIES), lambda i: (0, 0)),
            pl.BlockSpec((NUM_CATEGORIES, 1), lambda i: (0, 0)),
        ],
        out_specs=pl.BlockSpec((NUM_CATEGORIES, BM), lambda i: (0, i)),
        out_shape=jax.ShapeDtypeStruct((NUM_CATEGORIES, BATCH), jnp.float32),
    )(user_emb, cat_mean, W1u, W1c, b1, W2, b2)


def kernel(user_ids, category_ids, user_table, category_table, W1, b1, W2, b2):
    user_emb, cat_mean = _sc_fused(
        user_ids,
        user_table,
        category_ids.T,
        category_table.reshape(-1),
    )
    out_t = _mlp(
        user_emb,
        cat_mean,
        W1[:USER_DIM],
        W1[USER_DIM:],
        b1.reshape(-1, 1),
        W2,
        b2.reshape(-1, 1),
    )
    return out_t.T


# speedup vs baseline: 1.1202x; 1.1202x over previous
import functools

import jax
import jax.numpy as jnp
from jax import lax
from jax.experimental import pallas as pl
from jax.experimental.pallas import tpu as pltpu
from jax.experimental.pallas import tpu_sc as plsc

BATCH = 16384
HIST = 50
USER_DIM = 128
CAT_DIM = 64
NUM_CATEGORIES = 1000
DENSE_UNITS = 96

NC = 2
NS = 16
NW = NC * NS
BPW = BATCH // NW
LANES = 16
UCH = BPW // 4
NG = BPW // LANES


def _sc_fused(user_ids, user_table, cat_idx_flat, cat_table_flat):
    mesh = plsc.VectorSubcoreMesh(core_axis_name="c", subcore_axis_name="s")
    tab_words = (NUM_CATEGORIES + 1) * CAT_DIM

    @functools.partial(
        pl.kernel,
        out_type=(
            jax.ShapeDtypeStruct((BATCH, USER_DIM), jnp.float32),
            jax.ShapeDtypeStruct((BATCH, CAT_DIM), jnp.float32),
        ),
        mesh=mesh,
        scratch_types=[
            pltpu.VMEM((BPW,), jnp.int32),
            pltpu.VMEM((HIST, BPW), jnp.int32),
            pltpu.VMEM((tab_words,), jnp.float32),
            pltpu.VMEM((2, UCH, USER_DIM), jnp.float32),
            pltpu.VMEM((2, LANES, CAT_DIM), jnp.float32),
            pltpu.SemaphoreType.DMA,
            pltpu.SemaphoreType.DMA,
            pltpu.SemaphoreType.DMA,
        ],
        compiler_params=pltpu.CompilerParams(needs_layout_passes=False),
    )
    def k(uids_hbm, ut_hbm, cidx_hbm, ct_hbm, uout_hbm, cout_hbm,
          uid_v, cidx_v, tab_v, ubuf, obuf, gsem, wsem, csem):
        wid = lax.axis_index("s") * NC + lax.axis_index("c")
        base = wid * BPW
        pltpu.sync_copy(uids_hbm.at[pl.ds(base, BPW)], uid_v)

        def ug_desc(ch):
            return pltpu.make_async_copy(
                ut_hbm.at[uid_v.at[pl.ds(ch * UCH, UCH)]], ubuf.at[ch % 2], gsem)

        def uw_desc(ch):
            return pltpu.make_async_copy(
                ubuf.at[ch % 2], uout_hbm.at[pl.ds(base + ch * UCH, UCH)], wsem)

        ug_desc(0).start()
        pltpu.sync_copy(cidx_hbm.at[:, pl.ds(base, BPW)], cidx_v)
        pltpu.sync_copy(ct_hbm, tab_v)

        lane = lax.iota(jnp.int32, LANES)
        diag = tuple((lane + f0) & (LANES - 1) for f0 in range(LANES))
        inv = jnp.float32(1.0 / HIST)

        def service(ch):
            ug_desc(ch - 1).wait()
            uw_desc(ch - 1).start()
            if ch >= 2:
                uw_desc(ch - 2).wait()
            ug_desc(ch).start()

        def g_body(g, _):
            for ch in (1, 2, 3):
                pl.when(g == 8 * ch)(lambda ch=ch: service(ch))
            gbase = g * LANES
            pl.when(g >= 2)(
                lambda: pltpu.make_async_copy(
                    obuf.at[g % 2],
                    cout_hbm.at[pl.ds(base + gbase - 2 * LANES, LANES)],
                    csem,
                ).wait()
            )
            def dc_body(dc, _):
                foff = dc * LANES

                def h_body(h, acc):
                    c = cidx_v[h, pl.ds(gbase, LANES)]
                    b0 = c * CAT_DIM + foff
                    return tuple(
                        acc[f0] + plsc.load_gather(tab_v, [b0 + diag[f0]])
                        for f0 in range(LANES)
                    )

                acc0 = tuple(jnp.zeros((LANES,), jnp.float32) for _ in range(LANES))
                acc = lax.fori_loop(0, HIST, h_body, acc0)
                for f0 in range(LANES):
                    plsc.store_scatter(
                        obuf.at[g % 2], [lane, foff + diag[f0]], acc[f0] * inv)
                return 0

            lax.fori_loop(0, CAT_DIM // LANES, dc_body, 0)
            pltpu.async_copy(
                obuf.at[g % 2], cout_hbm.at[pl.ds(base + gbase, LANES)], csem)
            return 0

        lax.fori_loop(0, NG, g_body, 0)

        for g in (NG - 2, NG - 1):
            pltpu.make_async_copy(
                obuf.at[g % 2],
                cout_hbm.at[pl.ds(base + g * LANES, LANES)], csem).wait()
        ug_desc(3).wait()
        uw_desc(3).start()
        uw_desc(2).wait()
        uw_desc(3).wait()

    return k(user_ids, user_table, cat_idx_flat, cat_table_flat)


def _mlp(user_emb, cat_mean, W1u, W1c, b1, W2, b2):
    BM = 2048
    grid = (BATCH // BM,)

    def body(u_ref, c_ref, w1u_ref, w1c_ref, b1_ref, w2_ref, b2_ref, o_ref):
        xu = lax.dot_general(
            w1u_ref[...], u_ref[...], (((0,), (1,)), ((), ())),
            preferred_element_type=jnp.float32,
        )
        xc = lax.dot_general(
            w1c_ref[...], c_ref[...], (((0,), (1,)), ((), ())),
            preferred_element_type=jnp.float32,
        )
        x = jax.nn.relu(xu + xc + b1_ref[...])
        z = lax.dot_general(
            w2_ref[...], x, (((0,), (0,)), ((), ())),
            preferred_element_type=jnp.float32,
        )
        o_ref[...] = jax.nn.sigmoid(z + b2_ref[...])

    return pl.pallas_call(
        body,
        grid=grid,
        in_specs=[
            pl.BlockSpec((BM, USER_DIM), lambda i: (i, 0)),
            pl.BlockSpec((BM, CAT_DIM), lambda i: (i, 0)),
            pl.BlockSpec((USER_DIM, DENSE_UNITS), lambda i: (0, 0)),
            pl.BlockSpec((CAT_DIM, DENSE_UNITS), lambda i: (0, 0)),
            pl.BlockSpec((DENSE_UNITS, 1), lambda i: (0, 0)),
            pl.BlockSpec((DENSE_UNITS, NUM_CATEGORIES), lambda i: (0, 0)),
            pl.BlockSpec((NUM_CATEGORIES, 1), lambda i: (0, 0)),
        ],
        out_specs=pl.BlockSpec((NUM_CATEGORIES, BM), lambda i: (0, i)),
        out_shape=jax.ShapeDtypeStruct((NUM_CATEGORIES, BATCH), jnp.float32),
    )(user_emb, cat_mean, W1u, W1c, b1, W2, b2)


def kernel(user_ids, category_ids, user_table, category_table, W1, b1, W2, b2):
    user_emb, cat_mean = _sc_fused(
        user_ids,
        user_table,
        category_ids.T,
        category_table.reshape(-1),
    )
    out_t = _mlp(
        user_emb,
        cat_mean,
        W1[:USER_DIM],
        W1[USER_DIM:],
        b1.reshape(-1, 1),
        W2,
        b2.reshape(-1, 1),
    )
    return out_t.T
